# baseline (speedup 1.0000x reference)
import functools

import jax
import jax.numpy as jnp
from jax import lax
from jax.experimental import pallas as pl
from jax.experimental.pallas import tpu as pltpu
from jax.experimental.pallas import tpu_sc as plsc

_D = 1024
_NC = 2
_NS = 16
_NW = _NC * _NS
_CH = 32


def _make_lookup(seq, batch):
    nch = seq
    npairs = nch // 2

    mesh = plsc.VectorSubcoreMesh(core_axis_name="c", subcore_axis_name="s")

    @functools.partial(
        pl.kernel,
        out_type=(
            jax.ShapeDtypeStruct((seq * batch, _D), jnp.float32),
            jax.ShapeDtypeStruct((seq, batch), jnp.int32),
        ),
        mesh=mesh,
        scratch_types=[
            pltpu.VMEM((nch, batch), jnp.int32),
            pltpu.VMEM((batch,), jnp.int32),
            pltpu.VMEM((2, _CH, _D), jnp.float32),
            pltpu.SemaphoreType.DMA,
            pltpu.SemaphoreType.DMA,
        ],
    )
    def lookup(ids_hbm, mask_hbm, table_hbm, out_hbm, mask_out_hbm,
               idx_v, mask_v, bufs, sem0, sem1):
        wid = lax.axis_index("s") * _NC + lax.axis_index("c")
        col = wid * _CH
        pltpu.sync_copy(ids_hbm.at[pl.ds(0, nch)], idx_v)

        sems = (sem0, sem1)

        def gather(t, k):
            return pltpu.make_async_copy(
                table_hbm.at[idx_v.at[t, pl.ds(col, _CH)]], bufs.at[k], sems[k])

        def write_out(t, k):
            pltpu.sync_copy(bufs.at[k], out_hbm.at[pl.ds(t * batch + col, _CH)])

        gather(0, 0).start()

        @pl.loop(0, npairs)
        def _pair(p):
            t = p * 2
            gather(t + 1, 1).start()
            gather(t, 0).wait()
            write_out(t, 0)

            @pl.when(t + 2 < nch)
            def _():
                gather(t + 2, 0).start()

            gather(t + 1, 1).wait()
            write_out(t + 1, 1)

        pltpu.sync_copy(mask_hbm.at[wid], mask_v)
        pltpu.sync_copy(mask_v, mask_out_hbm.at[wid])

        @pl.when(wid + _NW < nch)
        def _():
            pltpu.sync_copy(mask_hbm.at[wid + _NW], mask_v)
            pltpu.sync_copy(mask_v, mask_out_hbm.at[wid + _NW])

    return lookup


def kernel(input_ids, attention_mask, table):
    batch, seq = input_ids.shape
    flat, mask_t = _make_lookup(seq, batch)(input_ids.T, attention_mask.T, table)
    emb = flat.reshape(seq, batch, _D).transpose(1, 0, 2)
    return emb, mask_t.T


# speedup vs baseline: 1.0209x; 1.0209x over previous
import functools

import jax
import jax.numpy as jnp
from jax import lax
from jax.experimental import pallas as pl
from jax.experimental.pallas import tpu as pltpu
from jax.experimental.pallas import tpu_sc as plsc

_D = 1024
_NC = 2
_NS = 16
_NW = _NC * _NS
_CH = 40
_NBUF = 3


def _make_lookup(n_rows):
    bpw = n_rows // _NW
    nch = bpw // _CH
    nloop = (nch - 1) // _NBUF

    mesh = plsc.VectorSubcoreMesh(core_axis_name="c", subcore_axis_name="s")

    @functools.partial(
        pl.kernel,
        out_type=jax.ShapeDtypeStruct((n_rows, _D), jnp.float32),
        mesh=mesh,
        scratch_types=[
            pltpu.VMEM((bpw,), jnp.int32),
            pltpu.VMEM((_NBUF, _CH, _D), jnp.float32),
            [pltpu.SemaphoreType.DMA] * _NBUF,
            [pltpu.SemaphoreType.DMA] * _NBUF,
        ],
    )
    def lookup(ids_hbm, table_hbm, out_hbm, idx_v, bufs, gsems, wsems):
        wid = lax.axis_index("s") * _NC + lax.axis_index("c")
        base = wid * bpw
        pltpu.sync_copy(ids_hbm.at[pl.ds(base, bpw)], idx_v)

        def gather(t, k):
            return pltpu.make_async_copy(
                table_hbm.at[idx_v.at[pl.ds(t * _CH, _CH)]],
                bufs.at[k],
                gsems[k],
            )

        def write(t, k):
            return pltpu.make_async_copy(
                bufs.at[k],
                out_hbm.at[pl.ds(base + t * _CH, _CH)],
                wsems[k],
            )

        gather(0, 0).start()
        gather(1, 1).start()

        @pl.loop(0, nloop)
        def _turn(p):
            s0 = p * _NBUF
            for r in range(_NBUF):
                s = s0 + r
                nxt = s + 2
                k2 = (r + 2) % _NBUF
                prev = nxt - _NBUF

                @pl.when(jnp.logical_and(prev >= 0, nxt < nch))
                def _():
                    write(prev, k2).wait()

                @pl.when(nxt < nch)
                def _():
                    gather(nxt, k2).start()

                gather(s, r).wait()
                write(s, r).start()

        for s in range(nloop * _NBUF, nch):
            k = s % _NBUF
            gather(s, k).wait()
            write(s, k).start()

        for s in range(nch - _NBUF, nch):
            write(s, s % _NBUF).wait()

    return lookup


def kernel(input_ids, attention_mask, table):
    batch, seq = input_ids.shape
    ids_flat = input_ids.T.reshape(batch * seq)
    flat = _make_lookup(batch * seq)(ids_flat, table)
    emb = flat.reshape(seq, batch, _D).transpose(1, 0, 2)
    return emb, attention_mask


# speedup vs baseline: 1.0432x; 1.0218x over previous
import functools

import jax
import jax.numpy as jnp
from jax import lax
from jax.experimental import pallas as pl
from jax.experimental.pallas import tpu as pltpu
from jax.experimental.pallas import tpu_sc as plsc

_D = 1024
_NC = 2
_NS = 16
_NW = _NC * _NS
_CH = 16


def _make_lookup(n_rows):
    bpw = n_rows // _NW
    nch = bpw // _CH

    mesh = plsc.VectorSubcoreMesh(core_axis_name="c", subcore_axis_name="s")

    @functools.partial(
        pl.kernel,
        out_type=jax.ShapeDtypeStruct((n_rows, _D), jnp.float32),
        mesh=mesh,
        scratch_types=[
            pltpu.VMEM((bpw,), jnp.int32),
            pltpu.VMEM((2, _CH, _D), jnp.float32),
            pltpu.VMEM_SHARED((_NS, 2, _CH, _D), jnp.float32),
            [pltpu.SemaphoreType.DMA] * 2,
            [pltpu.SemaphoreType.DMA] * 2,
        ],
    )
    def lookup(ids_hbm, table_hbm, out_hbm, idx_v, bufs, spm, gsems, wsems):
        cid = lax.axis_index("c")
        sid = lax.axis_index("s")
        wid = sid * _NC + cid
        base = wid * bpw
        pltpu.sync_copy(ids_hbm.at[pl.ds(base, bpw)], idx_v)

        def gather(t, k):
            return pltpu.make_async_copy(
                table_hbm.at[idx_v.at[pl.ds(t * _CH, _CH)]],
                bufs.at[k],
                gsems[k],
            )

        def write(t, k):
            return pltpu.make_async_copy(
                spm.at[sid, k],
                out_hbm.at[pl.ds(base + t * _CH, _CH)],
                wsems[k],
            )

        gather(0, 0).start()

        @pl.loop(0, nch // 2)
        def _pair(p):
            t0 = p * 2
            for k in (0, 1):
                t = t0 + k

                @pl.when(t + 1 < nch)
                def _():
                    gather(t + 1, 1 - k).start()

                gather(t, k).wait()

                @pl.when(t >= 2)
                def _():
                    write(t - 2, k).wait()

                pltpu.sync_copy(bufs.at[k], spm.at[sid, k])
                write(t, k).start()

        write(nch - 2, 0).wait()
        write(nch - 1, 1).wait()

    return lookup


def kernel(input_ids, attention_mask, table):
    batch, seq = input_ids.shape
    ids_flat = input_ids.T.reshape(batch * seq)
    flat = _make_lookup(batch * seq)(ids_flat, table)
    emb = flat.reshape(seq, batch, _D).transpose(1, 0, 2)
    return emb, attention_mask
